# Initial kernel scaffold; baseline (speedup 1.0000x reference)
import jax
import jax.numpy as jnp
from jax.experimental import pallas as pl


def kernel(x, edge_index, batch, W1, b1, W2, b2, W_out, b_out):
    raise NotImplementedError("write your pallas kernel here")


# speedup vs baseline: 22.8143x; 22.8143x over previous
import jax
import jax.numpy as jnp
from jax import lax
from jax.experimental import pallas as pl
from jax.experimental.pallas import tpu as pltpu
from jax.experimental.pallas import tpu_sc as plsc

NC = 2
NS = 16
NW = NC * NS
CHUNK = 128

_MESH = plsc.VectorSubcoreMesh(core_axis_name="c", subcore_axis_name="s")
_SC_PARAMS = pltpu.CompilerParams(use_tc_tiling_on_sc=False)


def _sc_degree(dst_p, zeros_vec, ones_vec, n_pad, chunks_pw):
    rows_pt = n_pad // NS
    e_pw = chunks_pw * CHUNK

    def body(dst_hbm, zero_hbm, ones_hbm, out_hbm, idx_v, ones_v, stage_v, deg_sh):
        cid = lax.axis_index("c")
        sid = lax.axis_index("s")
        base = (cid * NS + sid) * e_pw
        pltpu.sync_copy(zero_hbm.at[pl.ds(sid * rows_pt, rows_pt)], stage_v)
        pltpu.sync_copy(stage_v, deg_sh.at[pl.ds(sid * rows_pt, rows_pt)])
        pltpu.sync_copy(ones_hbm, ones_v)
        plsc.subcore_barrier()

        def step(j, carry):
            pltpu.sync_copy(dst_hbm.at[pl.ds(base + j * CHUNK, CHUNK)], idx_v)
            pltpu.sync_copy(ones_v, deg_sh.at[idx_v], add=True)
            return carry

        lax.fori_loop(0, chunks_pw, step, 0)
        plsc.subcore_barrier()
        pltpu.sync_copy(deg_sh.at[pl.ds(sid * rows_pt, rows_pt)], stage_v)
        pltpu.sync_copy(stage_v,
                        out_hbm.at[pl.ds(cid * n_pad + sid * rows_pt, rows_pt)])

    return pl.kernel(
        body,
        out_type=jax.ShapeDtypeStruct((NC * n_pad,), jnp.float32),
        mesh=_MESH,
        compiler_params=_SC_PARAMS,
        scratch_types=[
            pltpu.VMEM((CHUNK,), jnp.int32),
            pltpu.VMEM((CHUNK,), jnp.float32),
            pltpu.VMEM((rows_pt,), jnp.float32),
            pltpu.VMEM_SHARED((n_pad,), jnp.float32),
        ],
    )(dst_p, zeros_vec, ones_vec)


def _sc_scatter(hs, src_p, dst_p, zeros_mat, n_pad, d_hid, chunks_pw):
    rows_pt = n_pad // NS
    e_pw = chunks_pw * CHUNK

    def body(hs_hbm, src_hbm, dst_hbm, zero_hbm, out_hbm,
             sidx_v, didx_v, rows_v, stage_v, gsem, acc_sh):
        cid = lax.axis_index("c")
        sid = lax.axis_index("s")
        base = (cid * NS + sid) * e_pw
        pltpu.sync_copy(zero_hbm.at[pl.ds(sid * rows_pt, rows_pt)], stage_v)
        pltpu.sync_copy(stage_v, acc_sh.at[pl.ds(sid * rows_pt, rows_pt)])
        plsc.subcore_barrier()

        def step(j, carry):
            pltpu.sync_copy(src_hbm.at[pl.ds(base + j * CHUNK, CHUNK)], sidx_v)
            pltpu.sync_copy(dst_hbm.at[pl.ds(base + j * CHUNK, CHUNK)], didx_v)
            pltpu.async_copy(hs_hbm.at[sidx_v], rows_v, gsem).wait()
            pltpu.sync_copy(rows_v, acc_sh.at[didx_v], add=True)
            return carry

        lax.fori_loop(0, chunks_pw, step, 0)
        plsc.subcore_barrier()
        pltpu.sync_copy(acc_sh.at[pl.ds(sid * rows_pt, rows_pt)], stage_v)
        pltpu.sync_copy(stage_v,
                        out_hbm.at[pl.ds(cid * n_pad + sid * rows_pt, rows_pt)])

    return pl.kernel(
        body,
        out_type=jax.ShapeDtypeStruct((NC * n_pad, d_hid), jnp.float32),
        mesh=_MESH,
        compiler_params=_SC_PARAMS,
        scratch_types=[
            pltpu.VMEM((CHUNK,), jnp.int32),
            pltpu.VMEM((CHUNK,), jnp.int32),
            pltpu.VMEM((CHUNK, d_hid), jnp.float32),
            pltpu.VMEM((rows_pt, d_hid), jnp.float32),
            pltpu.SemaphoreType.DMA,
            pltpu.VMEM_SHARED((n_pad, d_hid), jnp.float32),
        ],
    )(hs, src_p, dst_p, zeros_mat)


def _tc_prescale(degp, x_p, W1, n, n_pad, d_hid):
    def body(degp_ref, x_ref, w_ref, hs_ref, dinv_ref):
        deg = degp_ref[0, :] + degp_ref[1, :] + 1.0
        rows = lax.broadcasted_iota(jnp.int32, (n_pad, 1), 0)
        dinv = jnp.where(rows < n, lax.rsqrt(deg)[:, None], 0.0)
        h = jnp.dot(x_ref[...], w_ref[...], preferred_element_type=jnp.float32)
        hs_ref[...] = h * dinv
        dinv_ref[...] = dinv

    return pl.pallas_call(
        body,
        out_shape=[jax.ShapeDtypeStruct((n_pad, d_hid), jnp.float32),
                   jax.ShapeDtypeStruct((n_pad, 1), jnp.float32)],
    )(degp, x_p, W1)


def _tc_mid(accp, hs1, dinv, W2, b1, n_pad, d_hid):
    def body(accp_ref, hs_ref, dinv_ref, w_ref, b_ref, out_ref):
        acc = accp_ref[0] + accp_ref[1]
        h1 = jnp.maximum((acc + hs_ref[...]) * dinv_ref[...] + b_ref[...], 0.0)
        out_ref[...] = jnp.dot(h1, w_ref[...],
                               preferred_element_type=jnp.float32) * dinv_ref[...]

    return pl.pallas_call(
        body,
        out_shape=jax.ShapeDtypeStruct((n_pad, d_hid), jnp.float32),
    )(accp, hs1, dinv, W2, b1)


def _tc_final(accp, hs2, dinv, b2, batch_p, W_out, b_out, n_pad, n_graphs):
    def body(accp_ref, hs_ref, dinv_ref, b2_ref, batch_ref, wo_ref, bo_ref, out_ref):
        acc = accp_ref[0] + accp_ref[1]
        h2 = jnp.maximum((acc + hs_ref[...]) * dinv_ref[...] + b2_ref[...], 0.0)
        seg = lax.broadcasted_iota(jnp.int32, (n_pad, n_graphs), 1)
        oh = (batch_ref[...] == seg).astype(jnp.float32)
        pooled = lax.dot_general(oh, h2, (((0,), (0,)), ((), ())),
                                 preferred_element_type=jnp.float32)
        out_ref[...] = jnp.dot(pooled, wo_ref[...],
                               preferred_element_type=jnp.float32) + bo_ref[...]

    return pl.pallas_call(
        body,
        out_shape=jax.ShapeDtypeStruct((n_graphs, W_out.shape[1]), jnp.float32),
    )(accp, hs2, dinv, b2, batch_p, W_out, b_out)


def kernel(x, edge_index, batch, W1, b1, W2, b2, W_out, b_out):
    n, d_in = x.shape
    d_hid = W1.shape[1]
    n_graphs = 64
    e = edge_index.shape[1]

    chunks_pw = -(-e // (NW * CHUNK))
    e_pad = NW * chunks_pw * CHUNK
    n_pad = -(-(n + 1) // (NS * 8)) * (NS * 8)

    src_p = jnp.concatenate([edge_index[0], jnp.full((e_pad - e,), n, jnp.int32)])
    dst_p = jnp.concatenate([edge_index[1], jnp.full((e_pad - e,), n, jnp.int32)])
    x_p = jnp.concatenate([x, jnp.zeros((n_pad - n, d_in), x.dtype)])
    batch_p = jnp.concatenate([batch, jnp.full((n_pad - n,), n_graphs, jnp.int32)])[:, None]
    zeros_vec = jnp.zeros((n_pad,), jnp.float32)
    zeros_mat = jnp.zeros((n_pad, d_hid), jnp.float32)
    ones_vec = jnp.ones((CHUNK,), jnp.float32)

    degp = _sc_degree(dst_p, zeros_vec, ones_vec, n_pad, chunks_pw).reshape(NC, n_pad)
    hs1, dinv = _tc_prescale(degp, x_p, W1, n, n_pad, d_hid)
    acc1 = _sc_scatter(hs1, src_p, dst_p, zeros_mat, n_pad, d_hid,
                       chunks_pw).reshape(NC, n_pad, d_hid)
    hs2 = _tc_mid(acc1, hs1, dinv, W2, b1.reshape(1, d_hid), n_pad, d_hid)
    acc2 = _sc_scatter(hs2, src_p, dst_p, zeros_mat, n_pad, d_hid,
                       chunks_pw).reshape(NC, n_pad, d_hid)
    return _tc_final(acc2, hs2, dinv, b2.reshape(1, d_hid), batch_p,
                     W_out, b_out.reshape(1, 1), n_pad, n_graphs)


# speedup vs baseline: 29.7336x; 1.3033x over previous
import jax
import jax.numpy as jnp
from jax import lax
from jax.experimental import pallas as pl
from jax.experimental.pallas import tpu as pltpu
from jax.experimental.pallas import tpu_sc as plsc

NC = 2
NS = 16
NW = NC * NS
CHUNK = 128
B = 8

_MESH = plsc.VectorSubcoreMesh(core_axis_name="c", subcore_axis_name="s")
_SC_PARAMS = pltpu.CompilerParams(use_tc_tiling_on_sc=False)


def _sc_degree(idx_pack, zeros_vec, ones_vec, n_pad, chunks_pw):
    rows_pt = n_pad // NS

    def body(idx_hbm, zero_hbm, ones_hbm, out_hbm, idx_v, ones_v, stage_v, sem, deg_sh):
        cid = lax.axis_index("c")
        sid = lax.axis_index("s")
        wid = cid * NS + sid
        pltpu.sync_copy(zero_hbm.at[pl.ds(sid * rows_pt, rows_pt)], stage_v)
        pltpu.sync_copy(stage_v, deg_sh.at[pl.ds(sid * rows_pt, rows_pt)])
        pltpu.sync_copy(ones_hbm, ones_v)
        plsc.subcore_barrier()

        def step(o, carry):
            row0 = (wid * chunks_pw + o * B) * 2
            pltpu.sync_copy(idx_hbm.at[pl.ds(row0, 2 * B)], idx_v)
            descs = [pltpu.async_copy(ones_v, deg_sh.at[idx_v.at[2 * b + 1]],
                                      sem, add=True) for b in range(B)]
            for d in descs:
                d.wait()
            return carry

        lax.fori_loop(0, chunks_pw // B, step, 0)
        plsc.subcore_barrier()
        pltpu.sync_copy(deg_sh.at[pl.ds(sid * rows_pt, rows_pt)], stage_v)
        pltpu.sync_copy(stage_v,
                        out_hbm.at[pl.ds(cid * n_pad + sid * rows_pt, rows_pt)])

    return pl.kernel(
        body,
        out_type=jax.ShapeDtypeStruct((NC * n_pad,), jnp.float32),
        mesh=_MESH,
        compiler_params=_SC_PARAMS,
        scratch_types=[
            pltpu.VMEM((2 * B, CHUNK), jnp.int32),
            pltpu.VMEM((CHUNK,), jnp.float32),
            pltpu.VMEM((rows_pt,), jnp.float32),
            pltpu.SemaphoreType.DMA,
            pltpu.VMEM_SHARED((n_pad,), jnp.float32),
        ],
    )(idx_pack, zeros_vec, ones_vec)


def _sc_scatter(hs, idx_pack, zeros_mat, n_pad, d_hid, chunks_pw):
    rows_pt = n_pad // NS

    def body(hs_hbm, idx_hbm, zero_hbm, out_hbm,
             idx_v, rows_v, stage_v, gsem, ssem, acc_sh):
        cid = lax.axis_index("c")
        sid = lax.axis_index("s")
        wid = cid * NS + sid
        pltpu.sync_copy(zero_hbm.at[pl.ds(sid * rows_pt, rows_pt)], stage_v)
        pltpu.sync_copy(stage_v, acc_sh.at[pl.ds(sid * rows_pt, rows_pt)])
        plsc.subcore_barrier()

        def step(o, carry):
            row0 = (wid * chunks_pw + o * B) * 2
            pltpu.sync_copy(idx_hbm.at[pl.ds(row0, 2 * B)], idx_v)
            gds = [pltpu.async_copy(hs_hbm.at[idx_v.at[2 * b]], rows_v.at[b], gsem)
                   for b in range(B)]
            for d in gds:
                d.wait()
            sds = [pltpu.async_copy(rows_v.at[b], acc_sh.at[idx_v.at[2 * b + 1]],
                                    ssem, add=True) for b in range(B)]
            for d in sds:
                d.wait()
            return carry

        lax.fori_loop(0, chunks_pw // B, step, 0)
        plsc.subcore_barrier()
        pltpu.sync_copy(acc_sh.at[pl.ds(sid * rows_pt, rows_pt)], stage_v)
        pltpu.sync_copy(stage_v,
                        out_hbm.at[pl.ds(cid * n_pad + sid * rows_pt, rows_pt)])

    return pl.kernel(
        body,
        out_type=jax.ShapeDtypeStruct((NC * n_pad, d_hid), jnp.float32),
        mesh=_MESH,
        compiler_params=_SC_PARAMS,
        scratch_types=[
            pltpu.VMEM((2 * B, CHUNK), jnp.int32),
            pltpu.VMEM((B, CHUNK, d_hid), jnp.float32),
            pltpu.VMEM((rows_pt, d_hid), jnp.float32),
            pltpu.SemaphoreType.DMA,
            pltpu.SemaphoreType.DMA,
            pltpu.VMEM_SHARED((n_pad, d_hid), jnp.float32),
        ],
    )(hs, idx_pack, zeros_mat)


def _tc_prescale(degp, x_p, W1, n, n_pad, d_hid):
    def body(degp_ref, x_ref, w_ref, hs_ref, dinv_ref):
        deg = degp_ref[0, :] + degp_ref[1, :] + 1.0
        rows = lax.broadcasted_iota(jnp.int32, (n_pad, 1), 0)
        dinv = jnp.where(rows < n, lax.rsqrt(deg)[:, None], 0.0)
        h = jnp.dot(x_ref[...], w_ref[...], preferred_element_type=jnp.float32)
        hs_ref[...] = h * dinv
        dinv_ref[...] = dinv

    return pl.pallas_call(
        body,
        out_shape=[jax.ShapeDtypeStruct((n_pad, d_hid), jnp.float32),
                   jax.ShapeDtypeStruct((n_pad, 1), jnp.float32)],
    )(degp, x_p, W1)


def _tc_mid(accp, hs1, dinv, W2, b1, n_pad, d_hid):
    def body(accp_ref, hs_ref, dinv_ref, w_ref, b_ref, out_ref):
        acc = accp_ref[0] + accp_ref[1]
        h1 = jnp.maximum((acc + hs_ref[...]) * dinv_ref[...] + b_ref[...], 0.0)
        out_ref[...] = jnp.dot(h1, w_ref[...],
                               preferred_element_type=jnp.float32) * dinv_ref[...]

    return pl.pallas_call(
        body,
        out_shape=jax.ShapeDtypeStruct((n_pad, d_hid), jnp.float32),
    )(accp, hs1, dinv, W2, b1)


def _tc_final(accp, hs2, dinv, b2, batch_p, W_out, b_out, n_pad, n_graphs):
    def body(accp_ref, hs_ref, dinv_ref, b2_ref, batch_ref, wo_ref, bo_ref, out_ref):
        acc = accp_ref[0] + accp_ref[1]
        h2 = jnp.maximum((acc + hs_ref[...]) * dinv_ref[...] + b2_ref[...], 0.0)
        seg = lax.broadcasted_iota(jnp.int32, (n_pad, n_graphs), 1)
        oh = (batch_ref[...] == seg).astype(jnp.float32)
        pooled = lax.dot_general(oh, h2, (((0,), (0,)), ((), ())),
                                 preferred_element_type=jnp.float32)
        out_ref[...] = jnp.dot(pooled, wo_ref[...],
                               preferred_element_type=jnp.float32) + bo_ref[...]

    return pl.pallas_call(
        body,
        out_shape=jax.ShapeDtypeStruct((n_graphs, W_out.shape[1]), jnp.float32),
    )(accp, hs2, dinv, b2, batch_p, W_out, b_out)


def kernel(x, edge_index, batch, W1, b1, W2, b2, W_out, b_out):
    n, d_in = x.shape
    d_hid = W1.shape[1]
    n_graphs = 64
    e = edge_index.shape[1]

    chunks_pw = -(-(-(-e // (NW * CHUNK))) // B) * B
    e_pad = NW * chunks_pw * CHUNK
    n_pad = -(-(n + 1) // (NS * 8)) * (NS * 8)

    src_p = jnp.concatenate([edge_index[0], jnp.full((e_pad - e,), n, jnp.int32)])
    dst_p = jnp.concatenate([edge_index[1], jnp.full((e_pad - e,), n, jnp.int32)])
    idx_pack = jnp.stack([src_p.reshape(-1, CHUNK), dst_p.reshape(-1, CHUNK)],
                         axis=1).reshape(-1, CHUNK)
    x_p = jnp.concatenate([x, jnp.zeros((n_pad - n, d_in), x.dtype)])
    batch_p = jnp.concatenate([batch, jnp.full((n_pad - n,), n_graphs, jnp.int32)])[:, None]
    zeros_vec = jnp.zeros((n_pad,), jnp.float32)
    zeros_mat = jnp.zeros((n_pad, d_hid), jnp.float32)
    ones_vec = jnp.ones((CHUNK,), jnp.float32)

    degp = _sc_degree(idx_pack, zeros_vec, ones_vec, n_pad, chunks_pw).reshape(NC, n_pad)
    hs1, dinv = _tc_prescale(degp, x_p, W1, n, n_pad, d_hid)
    acc1 = _sc_scatter(hs1, idx_pack, zeros_mat, n_pad, d_hid,
                       chunks_pw).reshape(NC, n_pad, d_hid)
    hs2 = _tc_mid(acc1, hs1, dinv, W2, b1.reshape(1, d_hid), n_pad, d_hid)
    acc2 = _sc_scatter(hs2, idx_pack, zeros_mat, n_pad, d_hid,
                       chunks_pw).reshape(NC, n_pad, d_hid)
    return _tc_final(acc2, hs2, dinv, b2.reshape(1, d_hid), batch_p,
                     W_out, b_out.reshape(1, 1), n_pad, n_graphs)


# speedup vs baseline: 63.0503x; 2.1205x over previous
import jax
import jax.numpy as jnp
from jax import lax
from jax.experimental import pallas as pl
from jax.experimental.pallas import tpu as pltpu
from jax.experimental.pallas import tpu_sc as plsc

NC = 2
NS = 16
NW = NC * NS
CHUNK = 128
B = 8

_MESH = plsc.VectorSubcoreMesh(core_axis_name="c", subcore_axis_name="s")
_SC_PARAMS = pltpu.CompilerParams(use_tc_tiling_on_sc=False)


def _sc_degree(idx_pack, zeros_vec, ones_vec, n_pad, chunks_pw):
    rows_pt = n_pad // NS

    def body(idx_hbm, zero_hbm, ones_hbm, out_hbm, idx_v, ones_v, stage_v, sem, deg_sh):
        cid = lax.axis_index("c")
        sid = lax.axis_index("s")
        wid = cid * NS + sid
        pltpu.sync_copy(zero_hbm.at[pl.ds(sid * rows_pt, rows_pt)], stage_v)
        pltpu.sync_copy(stage_v, deg_sh.at[pl.ds(sid * rows_pt, rows_pt)])
        pltpu.sync_copy(ones_hbm, ones_v)
        plsc.subcore_barrier()

        def step(o, carry):
            row0 = wid * chunks_pw + o * B
            pltpu.sync_copy(idx_hbm.at[pl.ds(row0, B)], idx_v)
            descs = [pltpu.async_copy(ones_v, deg_sh.at[idx_v.at[b]],
                                      sem, add=True) for b in range(B)]
            for d in descs:
                d.wait()
            return carry

        lax.fori_loop(0, chunks_pw // B, step, 0)
        plsc.subcore_barrier()
        pltpu.sync_copy(deg_sh.at[pl.ds(sid * rows_pt, rows_pt)], stage_v)
        pltpu.sync_copy(stage_v,
                        out_hbm.at[pl.ds(cid * n_pad + sid * rows_pt, rows_pt)])

    return pl.kernel(
        body,
        out_type=jax.ShapeDtypeStruct((NC * n_pad,), jnp.float32),
        mesh=_MESH,
        compiler_params=_SC_PARAMS,
        scratch_types=[
            pltpu.VMEM((B, CHUNK), jnp.int32),
            pltpu.VMEM((CHUNK,), jnp.float32),
            pltpu.VMEM((rows_pt,), jnp.float32),
            pltpu.SemaphoreType.DMA,
            pltpu.VMEM_SHARED((n_pad,), jnp.float32),
        ],
    )(idx_pack, zeros_vec, ones_vec)


def _sc_scatter(hs, idx_pack, zeros_mat, n_pad, d_hid, chunks_pw):
    rows_pt = n_pad // NS

    def body(hs_hbm, idx_hbm, zero_hbm, out_hbm,
             idx_v, rows_v, stage_v, gsem, ssem, acc_sh):
        cid = lax.axis_index("c")
        sid = lax.axis_index("s")
        wid = cid * NS + sid
        pltpu.sync_copy(zero_hbm.at[pl.ds(sid * rows_pt, rows_pt)], stage_v)
        pltpu.sync_copy(stage_v, acc_sh.at[pl.ds(sid * rows_pt, rows_pt)])
        plsc.subcore_barrier()

        def step(o, carry):
            row0 = (wid * chunks_pw + o * B) * 2
            pltpu.sync_copy(idx_hbm.at[pl.ds(row0, 2 * B)], idx_v)
            gds = [pltpu.async_copy(hs_hbm.at[idx_v.at[2 * b]], rows_v.at[b], gsem)
                   for b in range(B)]
            for d in gds:
                d.wait()
            sds = [pltpu.async_copy(rows_v.at[b], acc_sh.at[idx_v.at[2 * b + 1]],
                                    ssem, add=True) for b in range(B)]
            for d in sds:
                d.wait()
            return carry

        lax.fori_loop(0, chunks_pw // B, step, 0)
        plsc.subcore_barrier()
        pltpu.sync_copy(acc_sh.at[pl.ds(sid * rows_pt, rows_pt)], stage_v)
        pltpu.sync_copy(stage_v,
                        out_hbm.at[pl.ds(cid * n_pad + sid * rows_pt, rows_pt)])

    return pl.kernel(
        body,
        out_type=jax.ShapeDtypeStruct((NC * n_pad, d_hid), jnp.float32),
        mesh=_MESH,
        compiler_params=_SC_PARAMS,
        scratch_types=[
            pltpu.VMEM((2 * B, CHUNK), jnp.int32),
            pltpu.VMEM((B, CHUNK, d_hid), jnp.float32),
            pltpu.VMEM((rows_pt, d_hid), jnp.float32),
            pltpu.SemaphoreType.DMA,
            pltpu.SemaphoreType.DMA,
            pltpu.VMEM_SHARED((n_pad, d_hid), jnp.float32),
        ],
    )(hs, idx_pack, zeros_mat)


def _tc_prescale(degp, x_p, W1, n, n_pad, d_hid):
    def body(degp_ref, x_ref, w_ref, hs_ref, dinv_ref):
        deg = degp_ref[0, :] + degp_ref[1, :] + 1.0
        rows = lax.broadcasted_iota(jnp.int32, (n_pad, 1), 0)
        dinv = jnp.where(rows < n, lax.rsqrt(deg)[:, None], 0.0)
        h = jnp.dot(x_ref[...], w_ref[...], preferred_element_type=jnp.float32)
        hs_ref[...] = h * dinv
        dinv_ref[...] = dinv

    return pl.pallas_call(
        body,
        out_shape=[jax.ShapeDtypeStruct((n_pad, d_hid), jnp.float32),
                   jax.ShapeDtypeStruct((n_pad, 1), jnp.float32)],
    )(degp, x_p, W1)


def _tc_mid(accp, hs1, dinv, W2, b1, n_pad, d_hid):
    def body(accp_ref, hs_ref, dinv_ref, w_ref, b_ref, out_ref):
        acc = accp_ref[0] + accp_ref[1]
        h1 = jnp.maximum((acc + hs_ref[...]) * dinv_ref[...] + b_ref[...], 0.0)
        out_ref[...] = jnp.dot(h1, w_ref[...],
                               preferred_element_type=jnp.float32) * dinv_ref[...]

    return pl.pallas_call(
        body,
        out_shape=jax.ShapeDtypeStruct((n_pad, d_hid), jnp.float32),
    )(accp, hs1, dinv, W2, b1)


def _tc_final(accp, hs2, dinv, b2, batch_p, W_out, b_out, n_pad, n_graphs):
    def body(accp_ref, hs_ref, dinv_ref, b2_ref, batch_ref, wo_ref, bo_ref, out_ref):
        acc = accp_ref[0] + accp_ref[1]
        h2 = jnp.maximum((acc + hs_ref[...]) * dinv_ref[...] + b2_ref[...], 0.0)
        seg = lax.broadcasted_iota(jnp.int32, (n_pad, n_graphs), 1)
        oh = (batch_ref[...] == seg).astype(jnp.float32)
        pooled = lax.dot_general(oh, h2, (((0,), (0,)), ((), ())),
                                 preferred_element_type=jnp.float32)
        out_ref[...] = jnp.dot(pooled, wo_ref[...],
                               preferred_element_type=jnp.float32) + bo_ref[...]

    return pl.pallas_call(
        body,
        out_shape=jax.ShapeDtypeStruct((n_graphs, W_out.shape[1]), jnp.float32),
    )(accp, hs2, dinv, b2, batch_p, W_out, b_out)


def kernel(x, edge_index, batch, W1, b1, W2, b2, W_out, b_out):
    n, d_in = x.shape
    d_hid = W1.shape[1]
    n_graphs = 64
    e = edge_index.shape[1]

    chunks_pw = -(-(-(-e // (NW * CHUNK))) // B) * B
    e_pad = NW * chunks_pw * CHUNK
    n_pad = -(-(n + 1) // (NS * 8)) * (NS * 8)

    pad = jnp.arange(e_pad - e, dtype=jnp.int32)
    n_dummy = n_pad - n
    src_p = jnp.concatenate([edge_index[0], n + pad % n_dummy])
    dst_p = jnp.concatenate([edge_index[1], pad % n_pad])
    dstdeg_p = jnp.concatenate([edge_index[1], n + pad % n_dummy])
    idx_pack = jnp.stack([src_p.reshape(-1, CHUNK), dst_p.reshape(-1, CHUNK)],
                         axis=1).reshape(-1, CHUNK)
    deg_pack = dstdeg_p.reshape(-1, CHUNK)
    x_p = jnp.concatenate([x, jnp.zeros((n_pad - n, d_in), x.dtype)])
    batch_p = jnp.concatenate([batch, jnp.full((n_pad - n,), n_graphs, jnp.int32)])[:, None]
    zeros_vec = jnp.zeros((n_pad,), jnp.float32)
    zeros_mat = jnp.zeros((n_pad, d_hid), jnp.float32)
    ones_vec = jnp.ones((CHUNK,), jnp.float32)

    degp = _sc_degree(deg_pack, zeros_vec, ones_vec, n_pad, chunks_pw).reshape(NC, n_pad)
    hs1, dinv = _tc_prescale(degp, x_p, W1, n, n_pad, d_hid)
    acc1 = _sc_scatter(hs1, idx_pack, zeros_mat, n_pad, d_hid,
                       chunks_pw).reshape(NC, n_pad, d_hid)
    hs2 = _tc_mid(acc1, hs1, dinv, W2, b1.reshape(1, d_hid), n_pad, d_hid)
    acc2 = _sc_scatter(hs2, idx_pack, zeros_mat, n_pad, d_hid,
                       chunks_pw).reshape(NC, n_pad, d_hid)
    return _tc_final(acc2, hs2, dinv, b2.reshape(1, d_hid), batch_p,
                     W_out, b_out.reshape(1, 1), n_pad, n_graphs)


# speedup vs baseline: 86.5296x; 1.3724x over previous
import jax
import jax.numpy as jnp
from jax import lax
from jax.experimental import pallas as pl
from jax.experimental.pallas import tpu as pltpu
from jax.experimental.pallas import tpu_sc as plsc

NC = 2
NS = 16
NW = NC * NS
CHUNK = 128
B = 16

_MESH = plsc.VectorSubcoreMesh(core_axis_name="c", subcore_axis_name="s")
_SC_PARAMS = pltpu.CompilerParams(use_tc_tiling_on_sc=False)


def _sc_degree(dst2d, n_pad, chunks_pw):
    rows_pt = n_pad // NS
    n_outer = chunks_pw // B

    def body(dst_hbm, out_hbm, idx_v, ones_v, stage_v, isem, sem, deg_sh):
        cid = lax.axis_index("c")
        sid = lax.axis_index("s")
        wid = cid * NS + sid

        z16 = jnp.zeros((16,), jnp.float32)

        def zloop(r, carry):
            stage_v[pl.ds(r * 16, 16)] = z16
            return carry

        lax.fori_loop(0, rows_pt // 16, zloop, 0)
        for i in range(CHUNK // 16):
            ones_v[pl.ds(i * 16, 16)] = jnp.full((16,), 1.0, jnp.float32)
        pltpu.sync_copy(stage_v, deg_sh.at[pl.ds(sid * rows_pt, rows_pt)])

        def idx_rows(o):
            return pl.ds(wid * chunks_pw + o * B, B)

        pltpu.async_copy(dst_hbm.at[idx_rows(0)], idx_v.at[0], isem)
        plsc.subcore_barrier()

        def step(o, carry):
            slot = lax.rem(o, 2)
            pltpu.make_async_copy(dst_hbm.at[idx_rows(0)], idx_v.at[slot],
                                  isem).wait()

            @pl.when(o + 1 < n_outer)
            def _():
                pltpu.async_copy(dst_hbm.at[idx_rows(o + 1)], idx_v.at[1 - slot],
                                 isem)

            descs = [pltpu.async_copy(ones_v, deg_sh.at[idx_v.at[slot, b]],
                                      sem, add=True) for b in range(B)]
            for d in descs:
                d.wait()
            return carry

        lax.fori_loop(0, n_outer, step, 0)
        plsc.subcore_barrier()
        pltpu.sync_copy(deg_sh.at[pl.ds(sid * rows_pt, rows_pt)], stage_v)
        pltpu.sync_copy(stage_v,
                        out_hbm.at[pl.ds(cid * n_pad + sid * rows_pt, rows_pt)])

    return pl.kernel(
        body,
        out_type=jax.ShapeDtypeStruct((NC * n_pad,), jnp.float32),
        mesh=_MESH,
        compiler_params=_SC_PARAMS,
        scratch_types=[
            pltpu.VMEM((2, B, CHUNK), jnp.int32),
            pltpu.VMEM((CHUNK,), jnp.float32),
            pltpu.VMEM((rows_pt,), jnp.float32),
            pltpu.SemaphoreType.DMA,
            pltpu.SemaphoreType.DMA,
            pltpu.VMEM_SHARED((n_pad,), jnp.float32),
        ],
    )(dst2d)


def _sc_scatter(hs, src2d, dst2d, n_pad, d_hid, chunks_pw):
    rows_pt = n_pad // NS
    n_outer = chunks_pw // B

    def body(hs_hbm, src_hbm, dst_hbm, out_hbm,
             idx_v, rows_v, stage_v, isem, gsem, ssem, acc_sh):
        cid = lax.axis_index("c")
        sid = lax.axis_index("s")
        wid = cid * NS + sid

        z16 = jnp.zeros((16,), jnp.float32)

        def zloop(r, carry):
            stage_v[r, pl.ds(0, 16)] = z16
            stage_v[r, pl.ds(16, 16)] = z16
            return carry

        lax.fori_loop(0, rows_pt, zloop, 0)
        pltpu.sync_copy(stage_v, acc_sh.at[pl.ds(sid * rows_pt, rows_pt)])

        def idx_rows(o):
            return pl.ds(wid * chunks_pw + o * B, B)

        pltpu.async_copy(src_hbm.at[idx_rows(0)], idx_v.at[0, 0], isem)
        pltpu.async_copy(dst_hbm.at[idx_rows(0)], idx_v.at[0, 1], isem)
        plsc.subcore_barrier()

        def step(o, carry):
            slot = lax.rem(o, 2)
            pltpu.make_async_copy(src_hbm.at[idx_rows(0)], idx_v.at[slot, 0],
                                  isem).wait()
            pltpu.make_async_copy(dst_hbm.at[idx_rows(0)], idx_v.at[slot, 1],
                                  isem).wait()

            @pl.when(o + 1 < n_outer)
            def _():
                pltpu.async_copy(src_hbm.at[idx_rows(o + 1)],
                                 idx_v.at[1 - slot, 0], isem)
                pltpu.async_copy(dst_hbm.at[idx_rows(o + 1)],
                                 idx_v.at[1 - slot, 1], isem)

            gds = [pltpu.async_copy(hs_hbm.at[idx_v.at[slot, 0, b]],
                                    rows_v.at[b], gsem) for b in range(B)]
            sds = []
            for b in range(B):
                gds[b].wait()
                sds.append(pltpu.async_copy(rows_v.at[b],
                                            acc_sh.at[idx_v.at[slot, 1, b]],
                                            ssem, add=True))
            for d in sds:
                d.wait()
            return carry

        lax.fori_loop(0, n_outer, step, 0)
        plsc.subcore_barrier()
        pltpu.sync_copy(acc_sh.at[pl.ds(sid * rows_pt, rows_pt)], stage_v)
        pltpu.sync_copy(stage_v,
                        out_hbm.at[pl.ds(cid * n_pad + sid * rows_pt, rows_pt)])

    return pl.kernel(
        body,
        out_type=jax.ShapeDtypeStruct((NC * n_pad, d_hid), jnp.float32),
        mesh=_MESH,
        compiler_params=_SC_PARAMS,
        scratch_types=[
            pltpu.VMEM((2, 2, B, CHUNK), jnp.int32),
            pltpu.VMEM((B, CHUNK, d_hid), jnp.float32),
            pltpu.VMEM((rows_pt, d_hid), jnp.float32),
            pltpu.SemaphoreType.DMA,
            pltpu.SemaphoreType.DMA,
            pltpu.SemaphoreType.DMA,
            pltpu.VMEM_SHARED((n_pad, d_hid), jnp.float32),
        ],
    )(hs, src2d, dst2d)


def _tc_prescale(degp, x, W1, n, n_pad, d_hid, pad_edges):
    q, r = divmod(pad_edges, n_pad)

    def body(degp_ref, x_ref, w_ref, hs_ref, dinv_ref):
        rows = lax.broadcasted_iota(jnp.int32, (n_pad, 1), 0)
        contam = q + jnp.where(rows < r, 1.0, 0.0)
        deg = (degp_ref[pl.ds(0, n_pad)] + degp_ref[pl.ds(n_pad, n_pad)]
               )[:, None] + 1.0 - contam
        dinv = jnp.where(rows < n, lax.rsqrt(deg), 0.0)
        h = jnp.dot(x_ref[...], w_ref[...], preferred_element_type=jnp.float32)
        hs_ref[pl.ds(0, n), :] = h * dinv[:n, :]
        hs_ref[pl.ds(n, n_pad - n), :] = jnp.zeros((n_pad - n, d_hid), jnp.float32)
        dinv_ref[...] = dinv

    return pl.pallas_call(
        body,
        out_shape=[jax.ShapeDtypeStruct((n_pad, d_hid), jnp.float32),
                   jax.ShapeDtypeStruct((n_pad, 1), jnp.float32)],
    )(degp, x, W1)


def _tc_mid(accp, hs1, dinv, W2, b1, n_pad, d_hid):
    def body(accp_ref, hs_ref, dinv_ref, w_ref, b_ref, out_ref):
        acc = accp_ref[pl.ds(0, n_pad), :] + accp_ref[pl.ds(n_pad, n_pad), :]
        h1 = jnp.maximum((acc + hs_ref[...]) * dinv_ref[...] + b_ref[...], 0.0)
        out_ref[...] = jnp.dot(h1, w_ref[...],
                               preferred_element_type=jnp.float32) * dinv_ref[...]

    return pl.pallas_call(
        body,
        out_shape=jax.ShapeDtypeStruct((n_pad, d_hid), jnp.float32),
    )(accp, hs1, dinv, W2, b1)


def _tc_final(accp, hs2, dinv, b2, batch_p, W_out, b_out, n_pad, n_graphs):
    def body(accp_ref, hs_ref, dinv_ref, b2_ref, batch_ref, wo_ref, bo_ref, out_ref):
        acc = accp_ref[pl.ds(0, n_pad), :] + accp_ref[pl.ds(n_pad, n_pad), :]
        h2 = jnp.maximum((acc + hs_ref[...]) * dinv_ref[...] + b2_ref[...], 0.0)
        seg = lax.broadcasted_iota(jnp.int32, (n_pad, n_graphs), 1)
        oh = (batch_ref[...] == seg).astype(jnp.float32)
        pooled = lax.dot_general(oh, h2, (((0,), (0,)), ((), ())),
                                 preferred_element_type=jnp.float32)
        out_ref[...] = jnp.dot(pooled, wo_ref[...],
                               preferred_element_type=jnp.float32) + bo_ref[...]

    return pl.pallas_call(
        body,
        out_shape=jax.ShapeDtypeStruct((n_graphs, W_out.shape[1]), jnp.float32),
    )(accp, hs2, dinv, b2, batch_p, W_out, b_out)


def kernel(x, edge_index, batch, W1, b1, W2, b2, W_out, b_out):
    n, d_in = x.shape
    d_hid = W1.shape[1]
    n_graphs = 64
    e = edge_index.shape[1]

    chunks_pw = -(-(-(-e // (NW * CHUNK))) // B) * B
    e_pad = NW * chunks_pw * CHUNK
    n_pad = -(-(n + 1) // (NS * 16)) * (NS * 16)

    pad = jnp.arange(e_pad - e, dtype=jnp.int32)
    src2d = jnp.concatenate([edge_index[0],
                             n + pad % (n_pad - n)]).reshape(-1, CHUNK)
    dst2d = jnp.concatenate([edge_index[1], pad % n_pad]).reshape(-1, CHUNK)
    batch_p = jnp.concatenate(
        [batch, jnp.full((n_pad - n,), n_graphs, jnp.int32)])[:, None]

    degp = _sc_degree(dst2d, n_pad, chunks_pw)
    hs1, dinv = _tc_prescale(degp, x, W1, n, n_pad, d_hid, e_pad - e)
    acc1 = _sc_scatter(hs1, src2d, dst2d, n_pad, d_hid, chunks_pw)
    hs2 = _tc_mid(acc1, hs1, dinv, W2, b1.reshape(1, d_hid), n_pad, d_hid)
    acc2 = _sc_scatter(hs2, src2d, dst2d, n_pad, d_hid, chunks_pw)
    return _tc_final(acc2, hs2, dinv, b2.reshape(1, d_hid), batch_p,
                     W_out, b_out.reshape(1, 1), n_pad, n_graphs)


# speedup vs baseline: 88.2667x; 1.0201x over previous
import jax
import jax.numpy as jnp
from jax import lax
from jax.experimental import pallas as pl
from jax.experimental.pallas import tpu as pltpu
from jax.experimental.pallas import tpu_sc as plsc

NC = 2
NS = 16
NW = NC * NS
CHUNK = 128
B = 16
BE = 8

_MESH = plsc.VectorSubcoreMesh(core_axis_name="c", subcore_axis_name="s")
_SC_PARAMS = pltpu.CompilerParams(use_tc_tiling_on_sc=False)


def _sc_degree(dst2d, n_pad, chunks_pw):
    rows_pt = n_pad // NS
    n_outer = chunks_pw // B

    def body(dst_hbm, out_hbm, idx_v, ones_v, stage_v, isem, sem, deg_sh):
        cid = lax.axis_index("c")
        sid = lax.axis_index("s")
        wid = cid * NS + sid

        z16 = jnp.zeros((16,), jnp.float32)

        def zloop(r, carry):
            stage_v[pl.ds(r * 16, 16)] = z16
            return carry

        lax.fori_loop(0, rows_pt // 16, zloop, 0)
        for i in range(CHUNK // 16):
            ones_v[pl.ds(i * 16, 16)] = jnp.full((16,), 1.0, jnp.float32)
        pltpu.sync_copy(stage_v, deg_sh.at[pl.ds(sid * rows_pt, rows_pt)])

        def idx_rows(o):
            return pl.ds(wid * chunks_pw + o * B, B)

        pltpu.async_copy(dst_hbm.at[idx_rows(0)], idx_v.at[0], isem)
        plsc.subcore_barrier()

        def step(o, carry):
            slot = lax.rem(o, 2)
            pltpu.make_async_copy(dst_hbm.at[idx_rows(0)], idx_v.at[slot],
                                  isem).wait()

            @pl.when(o + 1 < n_outer)
            def _():
                pltpu.async_copy(dst_hbm.at[idx_rows(o + 1)], idx_v.at[1 - slot],
                                 isem)

            descs = [pltpu.async_copy(ones_v, deg_sh.at[idx_v.at[slot, b]],
                                      sem, add=True) for b in range(B)]
            for d in descs:
                d.wait()
            return carry

        lax.fori_loop(0, n_outer, step, 0)
        plsc.subcore_barrier()
        pltpu.sync_copy(deg_sh.at[pl.ds(sid * rows_pt, rows_pt)], stage_v)
        pltpu.sync_copy(stage_v,
                        out_hbm.at[pl.ds(cid * n_pad + sid * rows_pt, rows_pt)])

    return pl.kernel(
        body,
        out_type=jax.ShapeDtypeStruct((NC * n_pad,), jnp.float32),
        mesh=_MESH,
        compiler_params=_SC_PARAMS,
        scratch_types=[
            pltpu.VMEM((2, B, CHUNK), jnp.int32),
            pltpu.VMEM((CHUNK,), jnp.float32),
            pltpu.VMEM((rows_pt,), jnp.float32),
            pltpu.SemaphoreType.DMA,
            pltpu.SemaphoreType.DMA,
            pltpu.VMEM_SHARED((n_pad,), jnp.float32),
        ],
    )(dst2d)


def _sc_scatter(hs, src2d, dst2d, n_pad, d_hid, chunks_pw):
    rows_pt = n_pad // NS
    n_outer = chunks_pw // BE

    def body(hs_hbm, src_hbm, dst_hbm, out_hbm,
             idx_v, rows_v, stage_v, isem, gsem, ssem, acc_sh):
        cid = lax.axis_index("c")
        sid = lax.axis_index("s")
        wid = cid * NS + sid

        z16 = jnp.zeros((16,), jnp.float32)

        def zloop(r, carry):
            stage_v[r, pl.ds(0, 16)] = z16
            stage_v[r, pl.ds(16, 16)] = z16
            return carry

        lax.fori_loop(0, rows_pt, zloop, 0)
        pltpu.sync_copy(stage_v, acc_sh.at[pl.ds(sid * rows_pt, rows_pt)])

        def idx_rows(o):
            return pl.ds(wid * chunks_pw + o * BE, BE)

        def drain_scatters(slot):
            for b in range(BE):
                pltpu.make_async_copy(hs_hbm.at[pl.ds(0, CHUNK)],
                                      rows_v.at[slot, b], ssem).wait()

        pltpu.async_copy(src_hbm.at[idx_rows(0)], idx_v.at[0, 0], isem)
        pltpu.async_copy(dst_hbm.at[idx_rows(0)], idx_v.at[0, 1], isem)
        plsc.subcore_barrier()

        def step(o, carry):
            slot = lax.rem(o, 2)
            pltpu.make_async_copy(src_hbm.at[idx_rows(0)], idx_v.at[slot, 0],
                                  isem).wait()
            pltpu.make_async_copy(dst_hbm.at[idx_rows(0)], idx_v.at[slot, 1],
                                  isem).wait()

            @pl.when(o + 1 < n_outer)
            def _():
                pltpu.async_copy(src_hbm.at[idx_rows(o + 1)],
                                 idx_v.at[1 - slot, 0], isem)
                pltpu.async_copy(dst_hbm.at[idx_rows(o + 1)],
                                 idx_v.at[1 - slot, 1], isem)

            @pl.when(o >= 2)
            def _():
                drain_scatters(slot)

            gds = [pltpu.async_copy(hs_hbm.at[idx_v.at[slot, 0, b]],
                                    rows_v.at[slot, b], gsem) for b in range(BE)]
            for b in range(BE):
                gds[b].wait()
                pltpu.async_copy(rows_v.at[slot, b],
                                 acc_sh.at[idx_v.at[slot, 1, b]],
                                 ssem, add=True)
            return carry

        lax.fori_loop(0, n_outer, step, 0)
        drain_scatters(0)
        drain_scatters(1)
        plsc.subcore_barrier()
        pltpu.sync_copy(acc_sh.at[pl.ds(sid * rows_pt, rows_pt)], stage_v)
        pltpu.sync_copy(stage_v,
                        out_hbm.at[pl.ds(cid * n_pad + sid * rows_pt, rows_pt)])

    return pl.kernel(
        body,
        out_type=jax.ShapeDtypeStruct((NC * n_pad, d_hid), jnp.float32),
        mesh=_MESH,
        compiler_params=_SC_PARAMS,
        scratch_types=[
            pltpu.VMEM((2, 2, BE, CHUNK), jnp.int32),
            pltpu.VMEM((2, BE, CHUNK, d_hid), jnp.float32),
            pltpu.VMEM((rows_pt, d_hid), jnp.float32),
            pltpu.SemaphoreType.DMA,
            pltpu.SemaphoreType.DMA,
            pltpu.SemaphoreType.DMA,
            pltpu.VMEM_SHARED((n_pad, d_hid), jnp.float32),
        ],
    )(hs, src2d, dst2d)


def _tc_matmul(x, W1, n, d_hid):
    def body(x_ref, w_ref, h_ref):
        h_ref[...] = jnp.dot(x_ref[...], w_ref[...],
                             preferred_element_type=jnp.float32)

    return pl.pallas_call(
        body,
        out_shape=jax.ShapeDtypeStruct((n, d_hid), jnp.float32),
    )(x, W1)


def _tc_scale(degp, h1, n, n_pad, d_hid, pad_edges):
    q, r = divmod(pad_edges, n_pad)

    def body(degp_ref, h_ref, hs_ref, dinv_ref):
        rows = lax.broadcasted_iota(jnp.int32, (n_pad, 1), 0)
        contam = q + jnp.where(rows < r, 1.0, 0.0)
        deg = (degp_ref[pl.ds(0, n_pad)] + degp_ref[pl.ds(n_pad, n_pad)]
               )[:, None] + 1.0 - contam
        dinv = jnp.where(rows < n, lax.rsqrt(deg), 0.0)
        hs_ref[pl.ds(0, n), :] = h_ref[...] * dinv[:n, :]
        hs_ref[pl.ds(n, n_pad - n), :] = jnp.zeros((n_pad - n, d_hid), jnp.float32)
        dinv_ref[...] = dinv

    return pl.pallas_call(
        body,
        out_shape=[jax.ShapeDtypeStruct((n_pad, d_hid), jnp.float32),
                   jax.ShapeDtypeStruct((n_pad, 1), jnp.float32)],
    )(degp, h1)


def _tc_mid(accp, hs1, dinv, W2, b1, n_pad, d_hid):
    def body(accp_ref, hs_ref, dinv_ref, w_ref, b_ref, out_ref):
        acc = accp_ref[pl.ds(0, n_pad), :] + accp_ref[pl.ds(n_pad, n_pad), :]
        h1 = jnp.maximum((acc + hs_ref[...]) * dinv_ref[...] + b_ref[...], 0.0)
        out_ref[...] = jnp.dot(h1, w_ref[...],
                               preferred_element_type=jnp.float32) * dinv_ref[...]

    return pl.pallas_call(
        body,
        out_shape=jax.ShapeDtypeStruct((n_pad, d_hid), jnp.float32),
    )(accp, hs1, dinv, W2, b1)


def _tc_final(accp, hs2, dinv, b2, batch_p, W_out, b_out, n_pad, n_graphs):
    def body(accp_ref, hs_ref, dinv_ref, b2_ref, batch_ref, wo_ref, bo_ref, out_ref):
        acc = accp_ref[pl.ds(0, n_pad), :] + accp_ref[pl.ds(n_pad, n_pad), :]
        h2 = jnp.maximum((acc + hs_ref[...]) * dinv_ref[...] + b2_ref[...], 0.0)
        seg = lax.broadcasted_iota(jnp.int32, (n_pad, n_graphs), 1)
        oh = (batch_ref[...] == seg).astype(jnp.float32)
        pooled = lax.dot_general(oh, h2, (((0,), (0,)), ((), ())),
                                 preferred_element_type=jnp.float32)
        out_ref[...] = jnp.dot(pooled, wo_ref[...],
                               preferred_element_type=jnp.float32) + bo_ref[...]

    return pl.pallas_call(
        body,
        out_shape=jax.ShapeDtypeStruct((n_graphs, W_out.shape[1]), jnp.float32),
    )(accp, hs2, dinv, b2, batch_p, W_out, b_out)


def kernel(x, edge_index, batch, W1, b1, W2, b2, W_out, b_out):
    n, d_in = x.shape
    d_hid = W1.shape[1]
    n_graphs = 64
    e = edge_index.shape[1]

    chunks_pw = -(-(-(-e // (NW * CHUNK))) // B) * B
    e_pad = NW * chunks_pw * CHUNK
    n_pad = -(-(n + 1) // (NS * 16)) * (NS * 16)

    pad = jnp.arange(e_pad - e, dtype=jnp.int32)
    src2d = jnp.concatenate([edge_index[0],
                             n + pad % (n_pad - n)]).reshape(-1, CHUNK)
    dst2d = jnp.concatenate([edge_index[1], pad % n_pad]).reshape(-1, CHUNK)
    batch_p = jnp.concatenate(
        [batch, jnp.full((n_pad - n,), n_graphs, jnp.int32)])[:, None]

    degp = _sc_degree(dst2d, n_pad, chunks_pw)
    h1 = _tc_matmul(x, W1, n, d_hid)
    hs1, dinv = _tc_scale(degp, h1, n, n_pad, d_hid, e_pad - e)
    acc1 = _sc_scatter(hs1, src2d, dst2d, n_pad, d_hid, chunks_pw)
    hs2 = _tc_mid(acc1, hs1, dinv, W2, b1.reshape(1, d_hid), n_pad, d_hid)
    acc2 = _sc_scatter(hs2, src2d, dst2d, n_pad, d_hid, chunks_pw)
    return _tc_final(acc2, hs2, dinv, b2.reshape(1, d_hid), batch_p,
                     W_out, b_out.reshape(1, 1), n_pad, n_graphs)
